# baseline (speedup 1.0000x reference)
import functools

import jax
import jax.numpy as jnp
from jax.experimental import pallas as pl
from jax.experimental.pallas import tpu as pltpu
from jax.sharding import PartitionSpec as P

B = 4096
F = 100
D = 64
H = 512
NCORES = 2
BL = B // NCORES
BBL = 1024
NBL = BL // BBL
VOCAB_PAD = 128
EPS = 1e-5


def _stage0_kernel(xT_ref, embT_ref, W1_ref, b1_ref, h1T_ref, sums_ref,
                   dT0, dT1):
    t = pl.program_id(0)

    def gather_block(jb, dst):
        embT = embT_ref[...]
        for f in range(F):
            idx = xT_ref[pl.ds(f, 1), pl.ds(jb * BBL, BBL)]
            idxb = jnp.broadcast_to(idx, (D, BBL))
            g = jnp.take_along_axis(embT, idxb, axis=1)
            dst[pl.ds(f * D, D), :] = g.astype(jnp.bfloat16)

    def mm1_block(jb, src):
        h1 = jnp.dot(W1_ref[...], src[...],
                     preferred_element_type=jnp.float32)
        h1 = h1 + b1_ref[...]
        h1T_ref[:, pl.ds(jb * BBL, BBL)] = h1.astype(jnp.bfloat16)
        bs = jnp.sum(h1, axis=1, keepdims=True)
        bq = jnp.sum(h1 * h1, axis=1, keepdims=True)
        part = jnp.concatenate([bs, bq], axis=1)
        if jb == 0:
            sums_ref[...] = part
        else:
            sums_ref[...] += part

    @pl.when(t == 0)
    def _():
        gather_block(0, dT0)

    @pl.when(t == 1)
    def _():
        gather_block(1, dT1)
        mm1_block(0, dT0)

    @pl.when(t == 2)
    def _():
        mm1_block(1, dT1)


def _stage1_kernel(h1T_ref, sums_ref, W2_ref, g1_ref, be1_ref, b2_ref,
                   h2T_ref, sums2_ref):
    mu = sums_ref[:, 0:1] * (1.0 / B)
    var = sums_ref[:, 1:2] * (1.0 / B) - mu * mu
    rs = jax.lax.rsqrt(var + EPS)
    a = g1_ref[...] * rs
    c = be1_ref[...] - mu * a
    h1 = h1T_ref[...].astype(jnp.float32)
    nh = jnp.maximum(h1 * a + c, 0.0).astype(jnp.bfloat16)
    h2 = jnp.dot(W2_ref[...], nh,
                 preferred_element_type=jnp.float32) + b2_ref[...]
    h2T_ref[...] = h2.astype(jnp.bfloat16)
    bs = jnp.sum(h2, axis=1, keepdims=True)
    bq = jnp.sum(h2 * h2, axis=1, keepdims=True)
    sums2_ref[...] = jnp.concatenate([bs, bq], axis=1)


def _stage2_kernel(h2T_ref, sums_ref, xT_ref, W3_ref, wideW_ref,
                   g2_ref, be2_ref, c3_ref, out_ref):
    mu = sums_ref[:, 0:1] * (1.0 / B)
    var = sums_ref[:, 1:2] * (1.0 / B) - mu * mu
    rs = jax.lax.rsqrt(var + EPS)
    a = g2_ref[...] * rs
    c = be2_ref[...] - mu * a
    h2 = h2T_ref[...].astype(jnp.float32)
    nh = jnp.maximum(h2 * a + c, 0.0)
    logit = jnp.sum(nh * W3_ref[...], axis=0, keepdims=True)
    xf = xT_ref[...].astype(jnp.float32)
    wide = jnp.sum(xf * wideW_ref[...], axis=0, keepdims=True)
    z = logit + wide + c3_ref[...]
    out_ref[...] = jax.nn.sigmoid(z)


def _core_impl(x, embT, W1b, W2b, W3c, wideWc, b1c, g1c, be1c, b2c, g2c,
               be2c, c3):
    xT = x.astype(jnp.int32).T
    cp = pltpu.CompilerParams(vmem_limit_bytes=100 * 1024 * 1024)
    fullspec = lambda shape: pl.BlockSpec(shape, lambda t: tuple(
        0 for _ in shape))

    h1T, sums1 = pl.pallas_call(
        _stage0_kernel,
        grid=(NBL + 1,),
        in_specs=[fullspec((F, BL)), fullspec((D, VOCAB_PAD)),
                  fullspec((H, F * D)), fullspec((H, 1))],
        out_specs=[fullspec((H, BL)), fullspec((H, 2))],
        out_shape=[jax.ShapeDtypeStruct((H, BL), jnp.bfloat16),
                   jax.ShapeDtypeStruct((H, 2), jnp.float32)],
        scratch_shapes=[pltpu.VMEM((F * D, BBL), jnp.bfloat16),
                        pltpu.VMEM((F * D, BBL), jnp.bfloat16)],
        compiler_params=cp,
    )(xT, embT, W1b, b1c)
    sums1g = jax.lax.psum(sums1, 'b')

    h2T, sums2 = pl.pallas_call(
        _stage1_kernel,
        out_shape=[jax.ShapeDtypeStruct((H, BL), jnp.bfloat16),
                   jax.ShapeDtypeStruct((H, 2), jnp.float32)],
        compiler_params=cp,
    )(h1T, sums1g, W2b, g1c, be1c, b2c)
    sums2g = jax.lax.psum(sums2, 'b')

    out = pl.pallas_call(
        _stage2_kernel,
        out_shape=jax.ShapeDtypeStruct((1, BL), jnp.float32),
        compiler_params=cp,
    )(h2T, sums2g, xT, W3c, wideWc, g2c, be2c, c3)
    return out.reshape(BL, 1)


@functools.partial(jax.jit, static_argnames=())
def kernel(x, wide_w, wide_b, emb, W1, b1, g1, be1, W2, b2, g2, be2, W3, b3):
    embT = jnp.zeros((D, VOCAB_PAD), jnp.float32).at[:, :F].set(emb.T)
    W1b = W1.astype(jnp.bfloat16)
    W2b = W2.astype(jnp.bfloat16)
    W3c = W3.reshape(H, 1)
    wideWc = wide_w.reshape(F, 1)
    col = lambda v: v.reshape(-1, 1)
    c3 = (b3 + wide_b).reshape(1, 1)

    mesh = jax.make_mesh((NCORES,), ('b',))
    rep = P()
    in_specs = (P('b', None), rep, rep, rep, rep, rep, rep, rep, rep,
                rep, rep, rep, rep)
    args = (x, embT, W1b, W2b, W3c, wideWc, col(b1), col(g1), col(be1),
            col(b2), col(g2), col(be2), c3)
    args = tuple(
        jax.reshard(a, jax.sharding.NamedSharding(mesh, spec))
        for a, spec in zip(args, in_specs))
    impl = jax.shard_map(_core_impl, mesh=mesh, in_specs=in_specs,
                         out_specs=P('b', None), check_vma=False)
    return impl(*args)


# speedup vs baseline: 5.2965x; 5.2965x over previous
import functools
import jax
import jax.numpy as jnp
from jax.experimental import pallas as pl
from jax.experimental.pallas import tpu as pltpu

B = 4096
F = 100
D = 64
H = 512
BB = 2048
NB = B // BB
VOCAB_PAD = 128
KC = 4
FC = F // KC
EPS = 1e-5


def _wnd_kernel(xT_ref, xTn_ref, embT_ref, W1_ref, W2_ref, W3_ref, wideW_ref,
                b1_ref, g1_ref, be1_ref, b2_ref, g2_ref, be2_ref, c3_ref,
                out_ref, W1c, dT0, dT1, h1T, h2T, s1, q1, s2, q2):
    phase = pl.program_id(0)
    j = pl.program_id(1)

    def gather_chunk(x_ref, kf, dst):
        embT = embT_ref[...]
        for f2 in range(FC):
            f = kf * FC + f2
            idx = x_ref[pl.ds(f, 1), :]
            idxb = jnp.broadcast_to(idx, (D, BB))
            g = jnp.take_along_axis(embT, idxb, axis=1)
            dst[pl.ds(f2 * D, D), :] = g.astype(jnp.bfloat16)

    def stats_for(hT, jd, s, q):
        h = hT[jd].astype(jnp.float32)
        bs = jnp.sum(h, axis=1, keepdims=True)
        bq = jnp.sum(h * h, axis=1, keepdims=True)

        @pl.when(jd == 0)
        def _():
            s[...] = bs
            q[...] = bq

        @pl.when(jd > 0)
        def _():
            s[...] += bs
            q[...] += bq

    @pl.when(phase == 0)
    def _p0():
        @pl.when(j == 0)
        def _():
            W1c[...] = W1_ref[...].astype(jnp.bfloat16)
            gather_chunk(xT_ref, 0, dT0)

        bufs = (dT0, dT1)
        acc = jnp.broadcast_to(b1_ref[...], (H, BB))
        for k in range(KC):
            if k < KC - 1:
                gather_chunk(xT_ref, k + 1, bufs[(k + 1) % 2])
            else:
                gather_chunk(xTn_ref, 0, bufs[0])
            acc = acc + jnp.dot(W1c[:, pl.ds(k * FC * D, FC * D)],
                                bufs[k % 2][...],
                                preferred_element_type=jnp.float32)
        h1T[j] = acc.astype(jnp.bfloat16)

        @pl.when(j > 0)
        def _():
            stats_for(h1T, j - 1, s1, q1)

    @pl.when(phase == 1)
    def _p1():
        @pl.when(j == 0)
        def _():
            stats_for(h1T, NB - 1, s1, q1)

        mu = s1[...] * (1.0 / B)
        var = q1[...] * (1.0 / B) - mu * mu
        rs = jax.lax.rsqrt(var + EPS)
        a = g1_ref[...] * rs
        c = be1_ref[...] - mu * a
        h1 = h1T[j].astype(jnp.float32)
        nh = jnp.maximum(h1 * a + c, 0.0).astype(jnp.bfloat16)
        h2 = jnp.dot(W2_ref[...], nh,
                     preferred_element_type=jnp.float32) + b2_ref[...]
        h2T[j] = h2.astype(jnp.bfloat16)

        @pl.when(j > 0)
        def _():
            stats_for(h2T, j - 1, s2, q2)

    @pl.when(phase == 2)
    def _p2():
        @pl.when(j == 0)
        def _():
            stats_for(h2T, NB - 1, s2, q2)

        mu = s2[...] * (1.0 / B)
        var = q2[...] * (1.0 / B) - mu * mu
        rs = jax.lax.rsqrt(var + EPS)
        a = g2_ref[...] * rs
        c = be2_ref[...] - mu * a
        h2 = h2T[j].astype(jnp.float32)
        nh = jnp.maximum(h2 * a + c, 0.0)
        logit = jnp.sum(nh * W3_ref[...], axis=0, keepdims=True)
        xf = xT_ref[...].astype(jnp.float32)
        wide = jnp.sum(xf * wideW_ref[...], axis=0, keepdims=True)
        z = logit + wide + c3_ref[...]
        out_ref[...] = jax.nn.sigmoid(z).reshape(1, 1, BB)


@functools.partial(jax.jit, static_argnames=())
def kernel(x, wide_w, wide_b, emb, W1, b1, g1, be1, W2, b2, g2, be2, W3, b3):
    xT = x.astype(jnp.int32).T
    embT = jnp.zeros((D, VOCAB_PAD), jnp.float32).at[:, :F].set(emb.T)
    W2b = W2.astype(jnp.bfloat16)
    W3c = W3.reshape(H, 1)
    wideWc = wide_w.reshape(F, 1)
    col = lambda v: v.reshape(-1, 1)
    c3 = (b3 + wide_b).reshape(1, 1)

    full = lambda shape: pl.BlockSpec(shape, lambda p, j: (0, 0))
    grid = (3, NB)
    out = pl.pallas_call(
        _wnd_kernel,
        grid=grid,
        in_specs=[
            pl.BlockSpec((F, BB), lambda p, j: (0, j)),
            pl.BlockSpec((F, BB),
                         lambda p, j: (0, jnp.minimum(j + 1, NB - 1))),
            full((D, VOCAB_PAD)),
            full((H, F * D)),
            full((H, H)),
            full((H, 1)),
            full((F, 1)),
            full((H, 1)), full((H, 1)), full((H, 1)),
            full((H, 1)), full((H, 1)), full((H, 1)),
            full((1, 1)),
        ],
        out_specs=pl.BlockSpec((1, 1, BB), lambda p, j: (p, 0, j)),
        out_shape=jax.ShapeDtypeStruct((3, 1, B), jnp.float32),
        scratch_shapes=[
            pltpu.VMEM((H, F * D), jnp.bfloat16),
            pltpu.VMEM((FC * D, BB), jnp.bfloat16),
            pltpu.VMEM((FC * D, BB), jnp.bfloat16),
            pltpu.VMEM((NB, H, BB), jnp.bfloat16),
            pltpu.VMEM((NB, H, BB), jnp.bfloat16),
            pltpu.VMEM((H, 1), jnp.float32),
            pltpu.VMEM((H, 1), jnp.float32),
            pltpu.VMEM((H, 1), jnp.float32),
            pltpu.VMEM((H, 1), jnp.float32),
        ],
        compiler_params=pltpu.CompilerParams(
            dimension_semantics=("arbitrary", "arbitrary"),
            vmem_limit_bytes=100 * 1024 * 1024,
        ),
    )(xT, xT, embT, W1, W2b, W3c, wideWc,
      col(b1), col(g1), col(be1), col(b2), col(g2), col(be2), c3)
    return out[2].reshape(B, 1)


# speedup vs baseline: 5.4991x; 1.0383x over previous
import functools
import jax
import jax.numpy as jnp
from jax.experimental import pallas as pl
from jax.experimental.pallas import tpu as pltpu

B = 4096
F = 100
D = 64
H = 512
BB = 2048
NB = B // BB
VOCAB_PAD = 128
KC = 4
FC = F // KC
EPS = 1e-5


def _wnd_kernel(xT_ref, xTn_ref, embP_ref, W1_ref, W2_ref, W3_ref, wideW_ref,
                b1_ref, g1_ref, be1_ref, b2_ref, g2_ref, be2_ref, c3_ref,
                out_ref, W1c, dT0, dT1, h1T, h2T, s1, q1, s2, q2):
    phase = pl.program_id(0)
    j = pl.program_id(1)

    def gather_chunk(x_ref, kf, dst):
        embP = embP_ref[...]
        for f2 in range(FC):
            f = kf * FC + f2
            idx = x_ref[pl.ds(f, 1), :]
            idxb = jnp.broadcast_to(idx, (D // 2, BB))
            g = jnp.take_along_axis(embP, idxb, axis=1)
            dst[pl.ds(f2 * D, D), :] = pltpu.bitcast(g, jnp.bfloat16)

    def stats_for(hT, jd, s, q):
        h = hT[jd].astype(jnp.float32)
        bs = jnp.sum(h, axis=1, keepdims=True)
        bq = jnp.sum(h * h, axis=1, keepdims=True)

        @pl.when(jd == 0)
        def _():
            s[...] = bs
            q[...] = bq

        @pl.when(jd > 0)
        def _():
            s[...] += bs
            q[...] += bq

    @pl.when(phase == 0)
    def _p0():
        @pl.when(j == 0)
        def _():
            W1c[...] = W1_ref[...].astype(jnp.bfloat16)
            gather_chunk(xT_ref, 0, dT0)

        bufs = (dT0, dT1)
        acc = jnp.broadcast_to(b1_ref[...], (H, BB))
        for k in range(KC):
            if k < KC - 1:
                gather_chunk(xT_ref, k + 1, bufs[(k + 1) % 2])
            else:
                gather_chunk(xTn_ref, 0, bufs[0])
            acc = acc + jnp.dot(W1c[:, pl.ds(k * FC * D, FC * D)],
                                bufs[k % 2][...],
                                preferred_element_type=jnp.float32)
        h1T[j] = acc.astype(jnp.bfloat16)

        @pl.when(j > 0)
        def _():
            stats_for(h1T, j - 1, s1, q1)

    @pl.when(phase == 1)
    def _p1():
        @pl.when(j == 0)
        def _():
            stats_for(h1T, NB - 1, s1, q1)

        mu = s1[...] * (1.0 / B)
        var = q1[...] * (1.0 / B) - mu * mu
        rs = jax.lax.rsqrt(var + EPS)
        a = g1_ref[...] * rs
        c = be1_ref[...] - mu * a
        h1 = h1T[j].astype(jnp.float32)
        nh = jnp.maximum(h1 * a + c, 0.0).astype(jnp.bfloat16)
        h2 = jnp.dot(W2_ref[...], nh,
                     preferred_element_type=jnp.float32) + b2_ref[...]
        h2T[j] = h2.astype(jnp.bfloat16)

        @pl.when(j > 0)
        def _():
            stats_for(h2T, j - 1, s2, q2)

    @pl.when(phase == 2)
    def _p2():
        @pl.when(j == 0)
        def _():
            stats_for(h2T, NB - 1, s2, q2)

        mu = s2[...] * (1.0 / B)
        var = q2[...] * (1.0 / B) - mu * mu
        rs = jax.lax.rsqrt(var + EPS)
        a = g2_ref[...] * rs
        c = be2_ref[...] - mu * a
        h2 = h2T[j].astype(jnp.float32)
        nh = jnp.maximum(h2 * a + c, 0.0)
        logit = jnp.sum(nh * W3_ref[...], axis=0, keepdims=True)
        xf = xT_ref[...].astype(jnp.float32)
        wide = jnp.sum(xf * wideW_ref[...], axis=0, keepdims=True)
        z = logit + wide + c3_ref[...]
        out_ref[...] = jax.nn.sigmoid(z).reshape(1, 1, BB)


@functools.partial(jax.jit, static_argnames=())
def kernel(x, wide_w, wide_b, emb, W1, b1, g1, be1, W2, b2, g2, be2, W3, b3):
    xT = x.astype(jnp.int32).T
    eb = jnp.zeros((D, VOCAB_PAD), jnp.bfloat16).at[:, :F].set(
        emb.T.astype(jnp.bfloat16))
    eu = jax.lax.bitcast_convert_type(eb, jnp.uint16).astype(jnp.uint32)
    embP = jax.lax.bitcast_convert_type(
        eu[0::2, :] | (eu[1::2, :] << 16), jnp.int32)
    W2b = W2.astype(jnp.bfloat16)
    W3c = W3.reshape(H, 1)
    wideWc = wide_w.reshape(F, 1)
    col = lambda v: v.reshape(-1, 1)
    c3 = (b3 + wide_b).reshape(1, 1)

    full = lambda shape: pl.BlockSpec(shape, lambda p, j: (0, 0))
    grid = (3, NB)
    out = pl.pallas_call(
        _wnd_kernel,
        grid=grid,
        in_specs=[
            pl.BlockSpec((F, BB), lambda p, j: (0, j)),
            pl.BlockSpec((F, BB),
                         lambda p, j: (0, jnp.minimum(j + 1, NB - 1))),
            full((D // 2, VOCAB_PAD)),
            full((H, F * D)),
            full((H, H)),
            full((H, 1)),
            full((F, 1)),
            full((H, 1)), full((H, 1)), full((H, 1)),
            full((H, 1)), full((H, 1)), full((H, 1)),
            full((1, 1)),
        ],
        out_specs=pl.BlockSpec((1, 1, BB), lambda p, j: (p, 0, j)),
        out_shape=jax.ShapeDtypeStruct((3, 1, B), jnp.float32),
        scratch_shapes=[
            pltpu.VMEM((H, F * D), jnp.bfloat16),
            pltpu.VMEM((FC * D, BB), jnp.bfloat16),
            pltpu.VMEM((FC * D, BB), jnp.bfloat16),
            pltpu.VMEM((NB, H, BB), jnp.bfloat16),
            pltpu.VMEM((NB, H, BB), jnp.bfloat16),
            pltpu.VMEM((H, 1), jnp.float32),
            pltpu.VMEM((H, 1), jnp.float32),
            pltpu.VMEM((H, 1), jnp.float32),
            pltpu.VMEM((H, 1), jnp.float32),
        ],
        compiler_params=pltpu.CompilerParams(
            dimension_semantics=("arbitrary", "arbitrary"),
            vmem_limit_bytes=100 * 1024 * 1024,
        ),
    )(xT, xT, embP, W1, W2b, W3c, wideWc,
      col(b1), col(g1), col(be1), col(b2), col(g2), col(be2), c3)
    return out[2].reshape(B, 1)
